# Initial kernel scaffold; baseline (speedup 1.0000x reference)
import functools

import jax
import jax.numpy as jnp
from jax import lax
from jax.experimental import pallas as pl
from jax.experimental.pallas import tpu as pltpu
from jax.experimental.pallas import tpu_sc as plsc

N = 10000
E = 320000
D = 128
NUM_GRAPHS = 64

NCORES = 2
NSUB = 16
NTILES = NCORES * NSUB
EDGES_PER_TILE = E // NTILES
CHUNK = 80
NCHUNK = EDGES_PER_TILE // CHUNK
ROWS_PER_SUB = N // NSUB
NPAD = 10240
DEGW = 16
DROWS_PER_SUB = NPAD // NSUB

_mesh = plsc.VectorSubcoreMesh(core_axis_name="c", subcore_axis_name="s")


@functools.partial(
    pl.kernel,
    mesh=_mesh,
    out_type=(
        jax.ShapeDtypeStruct((NPAD, DEGW), jnp.float32),
        jax.ShapeDtypeStruct((NPAD, DEGW), jnp.float32),
    ),
    scratch_types=[
        pltpu.VMEM((CHUNK,), jnp.int32),
        pltpu.VMEM((CHUNK, DEGW), jnp.float32),
        pltpu.VMEM_SHARED((NPAD, DEGW), jnp.float32),
        pltpu.SemaphoreType.DMA,
    ],
)
def _count_kernel(dst_hbm, onerows_hbm, zrows_hbm, deg0_hbm, deg1_hbm,
                  didx, ones_v, deg_sp, sem):
    c = lax.axis_index("c")
    s = lax.axis_index("s")
    row0 = s * DROWS_PER_SUB
    pltpu.sync_copy(onerows_hbm, ones_v)
    pltpu.sync_copy(zrows_hbm, deg_sp.at[pl.ds(row0, DROWS_PER_SUB)])
    plsc.subcore_barrier()
    base = (c * NSUB + s) * EDGES_PER_TILE

    def body(k, carry):
        off = pl.multiple_of(base + k * CHUNK, 8)
        pltpu.sync_copy(dst_hbm.at[pl.ds(off, CHUNK)], didx)
        pltpu.sync_copy(ones_v, deg_sp.at[didx], add=True)
        return carry

    lax.fori_loop(0, NCHUNK, body, 0)
    plsc.subcore_barrier()

    @pl.when(c == 0)
    def _():
        pltpu.sync_copy(deg_sp.at[pl.ds(row0, DROWS_PER_SUB)],
                        deg0_hbm.at[pl.ds(row0, DROWS_PER_SUB)])

    @pl.when(c == 1)
    def _():
        pltpu.sync_copy(deg_sp.at[pl.ds(row0, DROWS_PER_SUB)],
                        deg1_hbm.at[pl.ds(row0, DROWS_PER_SUB)])


@functools.partial(
    pl.kernel,
    mesh=_mesh,
    out_type=(
        jax.ShapeDtypeStruct((N, D), jnp.float32),
        jax.ShapeDtypeStruct((N, D), jnp.float32),
    ),
    scratch_types=[
        pltpu.VMEM((CHUNK,), jnp.int32),
        pltpu.VMEM((CHUNK,), jnp.int32),
        pltpu.VMEM((CHUNK, D), jnp.float32),
        pltpu.VMEM_SHARED((N, D), jnp.float32),
        pltpu.SemaphoreType.DMA,
    ],
)
def _mp_kernel(y_hbm, src_hbm, dst_hbm, zrows_hbm, agg0_hbm, agg1_hbm,
               sidx, didx, rows_v, agg_sp, sem):
    c = lax.axis_index("c")
    s = lax.axis_index("s")
    row0 = s * ROWS_PER_SUB
    pltpu.sync_copy(zrows_hbm, agg_sp.at[pl.ds(row0, ROWS_PER_SUB)])
    plsc.subcore_barrier()
    base = (c * NSUB + s) * EDGES_PER_TILE

    def body(k, carry):
        off = pl.multiple_of(base + k * CHUNK, 8)
        pltpu.sync_copy(src_hbm.at[pl.ds(off, CHUNK)], sidx)
        pltpu.sync_copy(dst_hbm.at[pl.ds(off, CHUNK)], didx)
        pltpu.async_copy(y_hbm.at[sidx], rows_v, sem).wait()
        pltpu.sync_copy(rows_v, agg_sp.at[didx], add=True)
        return carry

    lax.fori_loop(0, NCHUNK, body, 0)
    plsc.subcore_barrier()

    @pl.when(c == 0)
    def _():
        pltpu.sync_copy(agg_sp.at[pl.ds(row0, ROWS_PER_SUB)],
                        agg0_hbm.at[pl.ds(row0, ROWS_PER_SUB)])

    @pl.when(c == 1)
    def _():
        pltpu.sync_copy(agg_sp.at[pl.ds(row0, ROWS_PER_SUB)],
                        agg1_hbm.at[pl.ds(row0, ROWS_PER_SUB)])


ROWS_BLK = 1250
GRID = N // ROWS_BLK


def _tc1_body(x_ref, w_ref, dinv_ref, y_ref):
    xw = jnp.dot(x_ref[...], w_ref[...], preferred_element_type=jnp.float32)
    y_ref[...] = xw * dinv_ref[...]


def _tc2_body(a0_ref, a1_ref, y1_ref, dinv_ref, b_ref, w_ref, y2_ref):
    di = dinv_ref[...]
    h = jnp.maximum(di * (a0_ref[...] + a1_ref[...] + y1_ref[...]) + b_ref[...], 0.0)
    y2_ref[...] = di * jnp.dot(h, w_ref[...], preferred_element_type=jnp.float32)


def _tc3_body(a0_ref, a1_ref, y2_ref, dinv_ref, b_ref, batch_ref, wl_ref,
              bl_ref, out_ref, psum_ref):
    i = pl.program_id(0)

    @pl.when(i == 0)
    def _():
        psum_ref[...] = jnp.zeros_like(psum_ref)

    di = dinv_ref[...]
    h = jnp.maximum(di * (a0_ref[...] + a1_ref[...] + y2_ref[...]) + b_ref[...], 0.0)
    gid = lax.broadcasted_iota(jnp.int32, (1, NUM_GRAPHS), 1)
    onehot = (batch_ref[...] == gid).astype(jnp.float32)
    hcat = jnp.concatenate([h, jnp.ones_like(h)], axis=1)
    psum_ref[...] += lax.dot_general(
        onehot, hcat, (((0,), (0,)), ((), ())),
        preferred_element_type=jnp.float32)

    @pl.when(i == GRID - 1)
    def _():
        cnt = jnp.maximum(psum_ref[:, D:D + 1], 1.0)
        pooled = psum_ref[:, :D] / cnt
        out_ref[...] = (jnp.dot(pooled, wl_ref[...],
                                preferred_element_type=jnp.float32)
                        + bl_ref[...])


def _row_spec(shape_tail):
    nz = len(shape_tail)
    return pl.BlockSpec((ROWS_BLK,) + shape_tail,
                        lambda i, _nz=nz: (i,) + (0,) * _nz)


def _full_spec(shape):
    nz = len(shape)
    return pl.BlockSpec(shape, lambda i, _nz=nz: (0,) * _nz)


_tc1 = pl.pallas_call(
    _tc1_body,
    grid=(GRID,),
    in_specs=[_row_spec((D,)), _full_spec((D, D)), _row_spec((1,))],
    out_specs=_row_spec((D,)),
    out_shape=jax.ShapeDtypeStruct((N, D), jnp.float32),
)

_tc2 = pl.pallas_call(
    _tc2_body,
    grid=(GRID,),
    in_specs=[_row_spec((D,)), _row_spec((D,)), _row_spec((D,)),
              _row_spec((1,)), _full_spec((1, D)), _full_spec((D, D))],
    out_specs=_row_spec((D,)),
    out_shape=jax.ShapeDtypeStruct((N, D), jnp.float32),
)

_tc3 = pl.pallas_call(
    _tc3_body,
    grid=(GRID,),
    in_specs=[_row_spec((D,)), _row_spec((D,)), _row_spec((D,)),
              _row_spec((1,)), _full_spec((1, D)), _row_spec((1,)),
              _full_spec((D, NUM_GRAPHS)), _full_spec((1, NUM_GRAPHS))],
    out_specs=_full_spec((NUM_GRAPHS, NUM_GRAPHS)),
    out_shape=jax.ShapeDtypeStruct((NUM_GRAPHS, NUM_GRAPHS), jnp.float32),
    scratch_shapes=[pltpu.VMEM((NUM_GRAPHS, 2 * D), jnp.float32)],
)


def kernel(x, edge_index, batch, W1, b1, W2, b2, Wl, bl):
    src = edge_index[0]
    dst = edge_index[1]

    onerows = jnp.zeros((CHUNK, DEGW), jnp.float32).at[:, 0].set(1.0)
    zrows_deg = jnp.zeros((DROWS_PER_SUB, DEGW), jnp.float32)
    zrows_agg = jnp.zeros((ROWS_PER_SUB, D), jnp.float32)

    deg0, deg1 = _count_kernel(dst, onerows, zrows_deg)
    deg = deg0[:N, 0] + deg1[:N, 0] + 1.0
    dinv = lax.rsqrt(deg).reshape(N, 1)

    y1 = _tc1(x, W1, dinv)
    a10, a11 = _mp_kernel(y1, src, dst, zrows_agg)
    y2 = _tc2(a10, a11, y1, dinv, b1.reshape(1, D), W2)
    a20, a21 = _mp_kernel(y2, src, dst, zrows_agg)
    out = _tc3(a20, a21, y2, dinv, b2.reshape(1, D),
               batch.reshape(N, 1), Wl, bl.reshape(1, NUM_GRAPHS))
    return out


# speedup vs baseline: 12.4991x; 12.4991x over previous
import functools

import jax
import jax.numpy as jnp
from jax import lax
from jax.experimental import pallas as pl
from jax.experimental.pallas import tpu as pltpu
from jax.experimental.pallas import tpu_sc as plsc

N = 10000
E = 320000
D = 128
NUM_GRAPHS = 64

NCORES = 2
NSUB = 16
NTILES = NCORES * NSUB
EDGES_PER_TILE = E // NTILES
CHUNK = 80
NCHUNK = EDGES_PER_TILE // CHUNK
NPAD = 10240
ROWS_PER_SUB = NPAD // NSUB
DEGW = 128
DROWS_PER_SUB = NPAD // NSUB

_mesh = plsc.VectorSubcoreMesh(core_axis_name="c", subcore_axis_name="s")


@functools.partial(
    pl.kernel,
    mesh=_mesh,
    out_type=jax.ShapeDtypeStruct((NCORES, NPAD, DEGW), jnp.float32),
    scratch_types=[
        pltpu.VMEM((CHUNK,), jnp.int32),
        pltpu.VMEM((CHUNK, DEGW), jnp.float32),
        pltpu.VMEM_SHARED((NPAD, DEGW), jnp.float32),
        pltpu.SemaphoreType.DMA,
    ],
)
def _count_kernel(dst_hbm, onerows_hbm, zrows_hbm, deg_hbm,
                  didx, ones_v, deg_sp, sem):
    c = lax.axis_index("c")
    s = lax.axis_index("s")
    row0 = s * DROWS_PER_SUB
    pltpu.sync_copy(onerows_hbm, ones_v)
    pltpu.sync_copy(zrows_hbm, deg_sp.at[pl.ds(row0, DROWS_PER_SUB)])
    plsc.subcore_barrier()
    base = (c * NSUB + s) * EDGES_PER_TILE

    def body(k, carry):
        off = pl.multiple_of(base + k * CHUNK, 8)
        pltpu.sync_copy(dst_hbm.at[pl.ds(off, CHUNK)], didx)
        pltpu.sync_copy(ones_v, deg_sp.at[didx], add=True)
        return carry

    lax.fori_loop(0, NCHUNK, body, 0)
    plsc.subcore_barrier()
    pltpu.sync_copy(deg_sp.at[pl.ds(row0, DROWS_PER_SUB)],
                    deg_hbm.at[c].at[pl.ds(row0, DROWS_PER_SUB)])


@functools.partial(
    pl.kernel,
    mesh=_mesh,
    out_type=jax.ShapeDtypeStruct((NCORES, NPAD, D), jnp.float32),
    scratch_types=[
        pltpu.VMEM((CHUNK,), jnp.int32),
        pltpu.VMEM((CHUNK,), jnp.int32),
        pltpu.VMEM((CHUNK, D), jnp.float32),
        pltpu.VMEM_SHARED((NPAD, D), jnp.float32),
        pltpu.SemaphoreType.DMA,
    ],
)
def _mp_kernel(y_hbm, src_hbm, dst_hbm, zrows_hbm, agg_hbm,
               sidx, didx, rows_v, agg_sp, sem):
    c = lax.axis_index("c")
    s = lax.axis_index("s")
    row0 = s * ROWS_PER_SUB
    pltpu.sync_copy(zrows_hbm, agg_sp.at[pl.ds(row0, ROWS_PER_SUB)])
    plsc.subcore_barrier()
    base = (c * NSUB + s) * EDGES_PER_TILE

    def body(k, carry):
        off = pl.multiple_of(base + k * CHUNK, 8)
        pltpu.sync_copy(src_hbm.at[pl.ds(off, CHUNK)], sidx)
        pltpu.sync_copy(dst_hbm.at[pl.ds(off, CHUNK)], didx)
        pltpu.async_copy(y_hbm.at[sidx], rows_v, sem).wait()
        pltpu.sync_copy(rows_v, agg_sp.at[didx], add=True)
        return carry

    lax.fori_loop(0, NCHUNK, body, 0)
    plsc.subcore_barrier()
    pltpu.sync_copy(agg_sp.at[pl.ds(row0, ROWS_PER_SUB)],
                    agg_hbm.at[c].at[pl.ds(row0, ROWS_PER_SUB)])


ROWS_BLK = 1000
GRID = N // ROWS_BLK


def _tc1_body(x_ref, w_ref, dinv_ref, y_ref):
    xw = jnp.dot(x_ref[...], w_ref[...], preferred_element_type=jnp.float32)
    y_ref[...] = xw * dinv_ref[...]


def _tc2_body(a_ref, y1_ref, dinv_ref, b_ref, w_ref, y2_ref):
    di = dinv_ref[...]
    h = jnp.maximum(di * (a_ref[0] + a_ref[1] + y1_ref[...]) + b_ref[...], 0.0)
    y2_ref[...] = di * jnp.dot(h, w_ref[...], preferred_element_type=jnp.float32)


def _tc3_body(a_ref, y2_ref, dinv_ref, b_ref, batch_ref, wl_ref,
              bl_ref, out_ref, psum_ref):
    i = pl.program_id(0)

    @pl.when(i == 0)
    def _():
        psum_ref[...] = jnp.zeros_like(psum_ref)

    di = dinv_ref[...]
    h = jnp.maximum(di * (a_ref[0] + a_ref[1] + y2_ref[...]) + b_ref[...], 0.0)
    gid = lax.broadcasted_iota(jnp.int32, (1, NUM_GRAPHS), 1)
    onehot = (batch_ref[...] == gid).astype(jnp.float32)
    hcat = jnp.concatenate([h, jnp.ones_like(h)], axis=1)
    psum_ref[...] += lax.dot_general(
        onehot, hcat, (((0,), (0,)), ((), ())),
        preferred_element_type=jnp.float32)

    @pl.when(i == GRID - 1)
    def _():
        cnt = jnp.maximum(psum_ref[:, D:D + 1], 1.0)
        pooled = psum_ref[:, :D] / cnt
        out_ref[...] = (jnp.dot(pooled, wl_ref[...],
                                preferred_element_type=jnp.float32)
                        + bl_ref[...])


def _row_spec(shape_tail):
    nz = len(shape_tail)
    return pl.BlockSpec((ROWS_BLK,) + shape_tail,
                        lambda i, _nz=nz: (i,) + (0,) * _nz)


_pair_spec = pl.BlockSpec((NCORES, ROWS_BLK, D), lambda i: (0, i, 0))


def _full_spec(shape):
    nz = len(shape)
    return pl.BlockSpec(shape, lambda i, _nz=nz: (0,) * _nz)


_tc1 = pl.pallas_call(
    _tc1_body,
    grid=(GRID,),
    in_specs=[_row_spec((D,)), _full_spec((D, D)), _row_spec((1,))],
    out_specs=_row_spec((D,)),
    out_shape=jax.ShapeDtypeStruct((N, D), jnp.float32),
)

_tc2 = pl.pallas_call(
    _tc2_body,
    grid=(GRID,),
    in_specs=[_pair_spec, _row_spec((D,)),
              _row_spec((1,)), _full_spec((1, D)), _full_spec((D, D))],
    out_specs=_row_spec((D,)),
    out_shape=jax.ShapeDtypeStruct((N, D), jnp.float32),
)

_tc3 = pl.pallas_call(
    _tc3_body,
    grid=(GRID,),
    in_specs=[_pair_spec, _row_spec((D,)),
              _row_spec((1,)), _full_spec((1, D)), _row_spec((1,)),
              _full_spec((D, NUM_GRAPHS)), _full_spec((1, NUM_GRAPHS))],
    out_specs=_full_spec((NUM_GRAPHS, NUM_GRAPHS)),
    out_shape=jax.ShapeDtypeStruct((NUM_GRAPHS, NUM_GRAPHS), jnp.float32),
    scratch_shapes=[pltpu.VMEM((NUM_GRAPHS, 2 * D), jnp.float32)],
)


def kernel(x, edge_index, batch, W1, b1, W2, b2, Wl, bl):
    src = edge_index[0]
    dst = edge_index[1]

    onerows = jnp.zeros((CHUNK, DEGW), jnp.float32).at[:, 0].set(1.0)
    zrows_deg = jnp.zeros((DROWS_PER_SUB, DEGW), jnp.float32)
    zrows_agg = jnp.zeros((ROWS_PER_SUB, D), jnp.float32)

    deg_pair = _count_kernel(dst, onerows, zrows_deg)
    deg = deg_pair[0, :N, 0] + deg_pair[1, :N, 0] + 1.0
    dinv = lax.rsqrt(deg).reshape(N, 1)

    y1 = _tc1(x, W1, dinv)
    a1 = _mp_kernel(y1, src, dst, zrows_agg)
    y2 = _tc2(a1, y1, dinv, b1.reshape(1, D), W2)
    a2 = _mp_kernel(y2, src, dst, zrows_agg)
    out = _tc3(a2, y2, dinv, b2.reshape(1, D),
               batch.reshape(N, 1), Wl, bl.reshape(1, NUM_GRAPHS))
    return out


# speedup vs baseline: 26.4431x; 2.1156x over previous
import functools

import jax
import jax.numpy as jnp
from jax import lax
from jax.experimental import pallas as pl
from jax.experimental.pallas import tpu as pltpu
from jax.experimental.pallas import tpu_sc as plsc

N = 10000
E = 320000
D = 128
NUM_GRAPHS = 64

NCORES = 2
NSUB = 16
NTILES = NCORES * NSUB
EDGES_PER_TILE = E // NTILES
CHUNK = 80
NCHUNK = EDGES_PER_TILE // CHUNK
NPAD = 10240
ROWS_PER_SUB = NPAD // NSUB
DEGW = 128
DROWS_PER_SUB = NPAD // NSUB

_mesh = plsc.VectorSubcoreMesh(core_axis_name="c", subcore_axis_name="s")


@functools.partial(
    pl.kernel,
    mesh=_mesh,
    out_type=jax.ShapeDtypeStruct((NCORES, NPAD, DEGW), jnp.float32),
    scratch_types=[
        pltpu.VMEM((NCHUNK, CHUNK), jnp.int32),
        pltpu.VMEM((CHUNK, DEGW), jnp.float32),
        pltpu.VMEM_SHARED((NPAD, DEGW), jnp.float32),
        pltpu.SemaphoreType.DMA,
    ],
)
def _count_kernel(dst3_hbm, onerows_hbm, zrows_hbm, deg_hbm,
                  didx, ones_v, deg_sp, sem):
    c = lax.axis_index("c")
    s = lax.axis_index("s")
    row0 = s * DROWS_PER_SUB
    wid = c * NSUB + s
    pltpu.sync_copy(onerows_hbm, ones_v)
    pltpu.sync_copy(dst3_hbm.at[wid], didx)
    pltpu.sync_copy(zrows_hbm, deg_sp.at[pl.ds(row0, DROWS_PER_SUB)])
    plsc.subcore_barrier()

    def body(k, carry):
        pltpu.sync_copy(ones_v, deg_sp.at[didx.at[k]], add=True)
        return carry

    lax.fori_loop(0, NCHUNK, body, 0)
    plsc.subcore_barrier()
    pltpu.sync_copy(deg_sp.at[pl.ds(row0, DROWS_PER_SUB)],
                    deg_hbm.at[c].at[pl.ds(row0, DROWS_PER_SUB)])


@functools.partial(
    pl.kernel,
    mesh=_mesh,
    out_type=jax.ShapeDtypeStruct((NCORES, NPAD, D), jnp.float32),
    scratch_types=[
        pltpu.VMEM((3, CHUNK), jnp.int32),
        pltpu.VMEM((3, CHUNK), jnp.int32),
        pltpu.VMEM((2, CHUNK, D), jnp.float32),
        pltpu.VMEM_SHARED((NPAD, D), jnp.float32),
        pltpu.SemaphoreType.DMA((3,)),
        pltpu.SemaphoreType.DMA((2,)),
    ],
)
def _mp_kernel(y_hbm, src_hbm, dst_hbm, zrows_hbm, agg_hbm,
               sidx, didx, rows_v, agg_sp, isem, gsem):
    c = lax.axis_index("c")
    s = lax.axis_index("s")
    row0 = s * ROWS_PER_SUB
    base = (c * NSUB + s) * EDGES_PER_TILE
    pltpu.sync_copy(zrows_hbm, agg_sp.at[pl.ds(row0, ROWS_PER_SUB)])

    def idx_load(k, j):
        off = pl.multiple_of(base + k * CHUNK, 8)
        pltpu.async_copy(src_hbm.at[pl.ds(off, CHUNK)], sidx.at[j], isem.at[j])
        pltpu.async_copy(dst_hbm.at[pl.ds(off, CHUNK)], didx.at[j], isem.at[j])

    def idx_wait(j):
        pltpu.make_async_copy(src_hbm.at[pl.ds(0, CHUNK)], sidx.at[j],
                              isem.at[j]).wait()
        pltpu.make_async_copy(dst_hbm.at[pl.ds(0, CHUNK)], didx.at[j],
                              isem.at[j]).wait()

    def gather_start(k, p):
        pltpu.async_copy(y_hbm.at[sidx.at[lax.rem(k, 3)]], rows_v.at[p],
                         gsem.at[p])

    idx_load(0, 0)
    idx_load(1, 1)
    plsc.subcore_barrier()
    idx_wait(0)
    gather_start(0, 0)

    def body(k, carry):
        p = lax.rem(k, 2)

        @pl.when(k + 2 < NCHUNK)
        def _():
            idx_load(k + 2, lax.rem(k + 2, 3))

        @pl.when(k + 1 < NCHUNK)
        def _():
            idx_wait(lax.rem(k + 1, 3))
            gather_start(k + 1, 1 - p)

        pltpu.make_async_copy(y_hbm.at[sidx.at[lax.rem(k, 3)]], rows_v.at[p],
                              gsem.at[p]).wait()
        pltpu.sync_copy(rows_v.at[p], agg_sp.at[didx.at[lax.rem(k, 3)]],
                        add=True)
        return carry

    lax.fori_loop(0, NCHUNK, body, 0)
    plsc.subcore_barrier()
    pltpu.sync_copy(agg_sp.at[pl.ds(row0, ROWS_PER_SUB)],
                    agg_hbm.at[c].at[pl.ds(row0, ROWS_PER_SUB)])


ROWS_BLK = 1000
GRID = N // ROWS_BLK


def _tc1_body(x_ref, w_ref, dinv_ref, y_ref):
    xw = jnp.dot(x_ref[...], w_ref[...], preferred_element_type=jnp.float32)
    y_ref[...] = xw * dinv_ref[...]


def _tc2_body(a_ref, y1_ref, dinv_ref, b_ref, w_ref, y2_ref):
    di = dinv_ref[...]
    h = jnp.maximum(di * (a_ref[0] + a_ref[1] + y1_ref[...]) + b_ref[...], 0.0)
    y2_ref[...] = di * jnp.dot(h, w_ref[...], preferred_element_type=jnp.float32)


def _tc3_body(a_ref, y2_ref, dinv_ref, b_ref, batch_ref, wl_ref,
              bl_ref, out_ref, psum_ref):
    i = pl.program_id(0)

    @pl.when(i == 0)
    def _():
        psum_ref[...] = jnp.zeros_like(psum_ref)

    di = dinv_ref[...]
    h = jnp.maximum(di * (a_ref[0] + a_ref[1] + y2_ref[...]) + b_ref[...], 0.0)
    gid = lax.broadcasted_iota(jnp.int32, (1, NUM_GRAPHS), 1)
    onehot = (batch_ref[...] == gid).astype(jnp.float32)
    hcat = jnp.concatenate([h, jnp.ones_like(h)], axis=1)
    psum_ref[...] += lax.dot_general(
        onehot, hcat, (((0,), (0,)), ((), ())),
        preferred_element_type=jnp.float32)

    @pl.when(i == GRID - 1)
    def _():
        cnt = jnp.maximum(psum_ref[:, D:D + 1], 1.0)
        pooled = psum_ref[:, :D] / cnt
        out_ref[...] = (jnp.dot(pooled, wl_ref[...],
                                preferred_element_type=jnp.float32)
                        + bl_ref[...])


def _row_spec(shape_tail):
    nz = len(shape_tail)
    return pl.BlockSpec((ROWS_BLK,) + shape_tail,
                        lambda i, _nz=nz: (i,) + (0,) * _nz)


_pair_spec = pl.BlockSpec((NCORES, ROWS_BLK, D), lambda i: (0, i, 0))


def _full_spec(shape):
    nz = len(shape)
    return pl.BlockSpec(shape, lambda i, _nz=nz: (0,) * _nz)


_tc1 = pl.pallas_call(
    _tc1_body,
    grid=(GRID,),
    in_specs=[_row_spec((D,)), _full_spec((D, D)), _row_spec((1,))],
    out_specs=_row_spec((D,)),
    out_shape=jax.ShapeDtypeStruct((N, D), jnp.float32),
)

_tc2 = pl.pallas_call(
    _tc2_body,
    grid=(GRID,),
    in_specs=[_pair_spec, _row_spec((D,)),
              _row_spec((1,)), _full_spec((1, D)), _full_spec((D, D))],
    out_specs=_row_spec((D,)),
    out_shape=jax.ShapeDtypeStruct((N, D), jnp.float32),
)

_tc3 = pl.pallas_call(
    _tc3_body,
    grid=(GRID,),
    in_specs=[_pair_spec, _row_spec((D,)),
              _row_spec((1,)), _full_spec((1, D)), _row_spec((1,)),
              _full_spec((D, NUM_GRAPHS)), _full_spec((1, NUM_GRAPHS))],
    out_specs=_full_spec((NUM_GRAPHS, NUM_GRAPHS)),
    out_shape=jax.ShapeDtypeStruct((NUM_GRAPHS, NUM_GRAPHS), jnp.float32),
    scratch_shapes=[pltpu.VMEM((NUM_GRAPHS, 2 * D), jnp.float32)],
)


def kernel(x, edge_index, batch, W1, b1, W2, b2, Wl, bl):
    src = edge_index[0]
    dst = edge_index[1]
    dst3 = dst.reshape(NTILES, NCHUNK, CHUNK)

    onerows = jnp.zeros((CHUNK, DEGW), jnp.float32).at[:, 0].set(1.0)
    zrows_deg = jnp.zeros((DROWS_PER_SUB, DEGW), jnp.float32)
    zrows_agg = jnp.zeros((ROWS_PER_SUB, D), jnp.float32)

    deg_pair = _count_kernel(dst3, onerows, zrows_deg)
    deg = deg_pair[0, :N, 0] + deg_pair[1, :N, 0] + 1.0
    dinv = lax.rsqrt(deg).reshape(N, 1)

    y1 = _tc1(x, W1, dinv)
    a1 = _mp_kernel(y1, src, dst, zrows_agg)
    y2 = _tc2(a1, y1, dinv, b1.reshape(1, D), W2)
    a2 = _mp_kernel(y2, src, dst, zrows_agg)
    out = _tc3(a2, y2, dinv, b2.reshape(1, D),
               batch.reshape(N, 1), Wl, bl.reshape(1, NUM_GRAPHS))
    return out


# speedup vs baseline: 26.5275x; 1.0032x over previous
import functools

import jax
import jax.numpy as jnp
from jax import lax
from jax.experimental import pallas as pl
from jax.experimental.pallas import tpu as pltpu
from jax.experimental.pallas import tpu_sc as plsc

N = 10000
E = 320000
D = 128
NUM_GRAPHS = 64

NCORES = 2
NSUB = 16
NTILES = NCORES * NSUB
EDGES_PER_TILE = E // NTILES
CHUNK = 80
NCHUNK = EDGES_PER_TILE // CHUNK
NPAD = 10240
ROWS_PER_SUB = NPAD // NSUB
DEGW = 128
DROWS_PER_SUB = NPAD // NSUB

_mesh = plsc.VectorSubcoreMesh(core_axis_name="c", subcore_axis_name="s")


@functools.partial(
    pl.kernel,
    mesh=_mesh,
    out_type=jax.ShapeDtypeStruct((NCORES, NPAD, DEGW), jnp.float32),
    scratch_types=[
        pltpu.VMEM((NCHUNK, CHUNK), jnp.int32),
        pltpu.VMEM((CHUNK, DEGW), jnp.float32),
        pltpu.VMEM_SHARED((NPAD, DEGW), jnp.float32),
        pltpu.SemaphoreType.DMA,
    ],
)
def _count_kernel(dst3_hbm, onerows_hbm, zrows_hbm, deg_hbm,
                  didx, ones_v, deg_sp, sem):
    c = lax.axis_index("c")
    s = lax.axis_index("s")
    row0 = s * DROWS_PER_SUB
    wid = c * NSUB + s
    pltpu.sync_copy(onerows_hbm, ones_v)
    pltpu.sync_copy(dst3_hbm.at[wid], didx)
    pltpu.sync_copy(zrows_hbm, deg_sp.at[pl.ds(row0, DROWS_PER_SUB)])
    plsc.subcore_barrier()

    def body(k, carry):
        pltpu.sync_copy(ones_v, deg_sp.at[didx.at[k]], add=True)
        return carry

    lax.fori_loop(0, NCHUNK, body, 0)
    plsc.subcore_barrier()
    pltpu.sync_copy(deg_sp.at[pl.ds(row0, DROWS_PER_SUB)],
                    deg_hbm.at[c].at[pl.ds(row0, DROWS_PER_SUB)])


@functools.partial(
    pl.kernel,
    mesh=_mesh,
    out_type=jax.ShapeDtypeStruct((NCORES, NPAD, D), jnp.float32),
    scratch_types=[
        pltpu.VMEM((3, CHUNK), jnp.int32),
        pltpu.VMEM((3, CHUNK), jnp.int32),
        pltpu.VMEM((2, CHUNK, D), jnp.float32),
        pltpu.VMEM_SHARED((NPAD, D), jnp.float32),
        pltpu.SemaphoreType.DMA((3,)),
        pltpu.SemaphoreType.DMA((2,)),
        pltpu.SemaphoreType.DMA((2,)),
    ],
)
def _mp_kernel(y_hbm, src_hbm, dst_hbm, zrows_hbm, agg_hbm,
               sidx, didx, rows_v, agg_sp, isem, gsem, ssem):
    c = lax.axis_index("c")
    s = lax.axis_index("s")
    row0 = s * ROWS_PER_SUB
    base = (c * NSUB + s) * EDGES_PER_TILE
    pltpu.sync_copy(zrows_hbm, agg_sp.at[pl.ds(row0, ROWS_PER_SUB)])

    def idx_load(k, j):
        off = pl.multiple_of(base + k * CHUNK, 8)
        pltpu.async_copy(src_hbm.at[pl.ds(off, CHUNK)], sidx.at[j], isem.at[j])
        pltpu.async_copy(dst_hbm.at[pl.ds(off, CHUNK)], didx.at[j], isem.at[j])

    def idx_wait(j):
        pltpu.make_async_copy(src_hbm.at[pl.ds(0, CHUNK)], sidx.at[j],
                              isem.at[j]).wait()
        pltpu.make_async_copy(dst_hbm.at[pl.ds(0, CHUNK)], didx.at[j],
                              isem.at[j]).wait()

    def gather_start(k, p):
        pltpu.async_copy(y_hbm.at[sidx.at[lax.rem(k, 3)]], rows_v.at[p],
                         gsem.at[p])

    def scatter_start(k, p):
        pltpu.async_copy(rows_v.at[p], agg_sp.at[didx.at[lax.rem(k, 3)]],
                         ssem.at[p], add=True)

    def scatter_wait(k, p):
        pltpu.make_async_copy(rows_v.at[p], agg_sp.at[didx.at[lax.rem(k, 3)]],
                              ssem.at[p]).wait()

    idx_load(0, 0)
    idx_load(1, 1)
    plsc.subcore_barrier()
    idx_wait(0)
    gather_start(0, 0)

    def body(k, carry):
        p = lax.rem(k, 2)

        @pl.when(k >= 1)
        def _():
            scatter_wait(k - 1, 1 - p)

        @pl.when(k + 2 < NCHUNK)
        def _():
            idx_load(k + 2, lax.rem(k + 2, 3))

        @pl.when(k + 1 < NCHUNK)
        def _():
            idx_wait(lax.rem(k + 1, 3))
            gather_start(k + 1, 1 - p)

        pltpu.make_async_copy(y_hbm.at[sidx.at[lax.rem(k, 3)]], rows_v.at[p],
                              gsem.at[p]).wait()
        scatter_start(k, p)
        return carry

    lax.fori_loop(0, NCHUNK, body, 0)
    scatter_wait(NCHUNK - 1, (NCHUNK - 1) % 2)
    plsc.subcore_barrier()
    pltpu.sync_copy(agg_sp.at[pl.ds(row0, ROWS_PER_SUB)],
                    agg_hbm.at[c].at[pl.ds(row0, ROWS_PER_SUB)])


ROWS_BLK = 1000
GRID = N // ROWS_BLK


def _tc1_body(x_ref, w_ref, dinv_ref, y_ref):
    xw = jnp.dot(x_ref[...], w_ref[...], preferred_element_type=jnp.float32)
    y_ref[...] = xw * dinv_ref[...]


def _tc2_body(a_ref, y1_ref, dinv_ref, b_ref, w_ref, y2_ref):
    di = dinv_ref[...]
    h = jnp.maximum(di * (a_ref[0] + a_ref[1] + y1_ref[...]) + b_ref[...], 0.0)
    y2_ref[...] = di * jnp.dot(h, w_ref[...], preferred_element_type=jnp.float32)


def _tc3_body(a_ref, y2_ref, dinv_ref, b_ref, batch_ref, wl_ref,
              bl_ref, out_ref, psum_ref):
    i = pl.program_id(0)

    @pl.when(i == 0)
    def _():
        psum_ref[...] = jnp.zeros_like(psum_ref)

    di = dinv_ref[...]
    h = jnp.maximum(di * (a_ref[0] + a_ref[1] + y2_ref[...]) + b_ref[...], 0.0)
    gid = lax.broadcasted_iota(jnp.int32, (1, NUM_GRAPHS), 1)
    onehot = (batch_ref[...] == gid).astype(jnp.float32)
    hcat = jnp.concatenate([h, jnp.ones_like(h)], axis=1)
    psum_ref[...] += lax.dot_general(
        onehot, hcat, (((0,), (0,)), ((), ())),
        preferred_element_type=jnp.float32)

    @pl.when(i == GRID - 1)
    def _():
        cnt = jnp.maximum(psum_ref[:, D:D + 1], 1.0)
        pooled = psum_ref[:, :D] / cnt
        out_ref[...] = (jnp.dot(pooled, wl_ref[...],
                                preferred_element_type=jnp.float32)
                        + bl_ref[...])


def _row_spec(shape_tail):
    nz = len(shape_tail)
    return pl.BlockSpec((ROWS_BLK,) + shape_tail,
                        lambda i, _nz=nz: (i,) + (0,) * _nz)


_pair_spec = pl.BlockSpec((NCORES, ROWS_BLK, D), lambda i: (0, i, 0))


def _full_spec(shape):
    nz = len(shape)
    return pl.BlockSpec(shape, lambda i, _nz=nz: (0,) * _nz)


_tc1 = pl.pallas_call(
    _tc1_body,
    grid=(GRID,),
    in_specs=[_row_spec((D,)), _full_spec((D, D)), _row_spec((1,))],
    out_specs=_row_spec((D,)),
    out_shape=jax.ShapeDtypeStruct((N, D), jnp.float32),
)

_tc2 = pl.pallas_call(
    _tc2_body,
    grid=(GRID,),
    in_specs=[_pair_spec, _row_spec((D,)),
              _row_spec((1,)), _full_spec((1, D)), _full_spec((D, D))],
    out_specs=_row_spec((D,)),
    out_shape=jax.ShapeDtypeStruct((N, D), jnp.float32),
)

_tc3 = pl.pallas_call(
    _tc3_body,
    grid=(GRID,),
    in_specs=[_pair_spec, _row_spec((D,)),
              _row_spec((1,)), _full_spec((1, D)), _row_spec((1,)),
              _full_spec((D, NUM_GRAPHS)), _full_spec((1, NUM_GRAPHS))],
    out_specs=_full_spec((NUM_GRAPHS, NUM_GRAPHS)),
    out_shape=jax.ShapeDtypeStruct((NUM_GRAPHS, NUM_GRAPHS), jnp.float32),
    scratch_shapes=[pltpu.VMEM((NUM_GRAPHS, 2 * D), jnp.float32)],
)


def kernel(x, edge_index, batch, W1, b1, W2, b2, Wl, bl):
    src = edge_index[0]
    dst = edge_index[1]
    dst3 = dst.reshape(NTILES, NCHUNK, CHUNK)

    onerows = jnp.zeros((CHUNK, DEGW), jnp.float32).at[:, 0].set(1.0)
    zrows_deg = jnp.zeros((DROWS_PER_SUB, DEGW), jnp.float32)
    zrows_agg = jnp.zeros((ROWS_PER_SUB, D), jnp.float32)

    deg_pair = _count_kernel(dst3, onerows, zrows_deg)
    deg = deg_pair[0, :N, 0] + deg_pair[1, :N, 0] + 1.0
    dinv = lax.rsqrt(deg).reshape(N, 1)

    y1 = _tc1(x, W1, dinv)
    a1 = _mp_kernel(y1, src, dst, zrows_agg)
    y2 = _tc2(a1, y1, dinv, b1.reshape(1, D), W2)
    a2 = _mp_kernel(y2, src, dst, zrows_agg)
    out = _tc3(a2, y2, dinv, b2.reshape(1, D),
               batch.reshape(N, 1), Wl, bl.reshape(1, NUM_GRAPHS))
    return out


# speedup vs baseline: 26.6111x; 1.0032x over previous
import functools

import jax
import jax.numpy as jnp
from jax import lax
from jax.experimental import pallas as pl
from jax.experimental.pallas import tpu as pltpu
from jax.experimental.pallas import tpu_sc as plsc

N = 10000
E = 320000
D = 128
NUM_GRAPHS = 64

NCORES = 2
NSUB = 16
NTILES = NCORES * NSUB
EDGES_PER_TILE = E // NTILES
CHUNK = 80
NCHUNK = EDGES_PER_TILE // CHUNK
NPAD = 10240
ROWS_PER_SUB = NPAD // NSUB
DEGW = 128
DROWS_PER_SUB = NPAD // NSUB

_mesh = plsc.VectorSubcoreMesh(core_axis_name="c", subcore_axis_name="s")


@functools.partial(
    pl.kernel,
    mesh=_mesh,
    out_type=jax.ShapeDtypeStruct((NCORES, NPAD, DEGW), jnp.float32),
    scratch_types=[
        pltpu.VMEM((NCHUNK, CHUNK), jnp.int32),
        pltpu.VMEM((CHUNK, DEGW), jnp.float32),
        pltpu.VMEM_SHARED((NPAD, DEGW), jnp.float32),
        pltpu.SemaphoreType.DMA((2,)),
    ],
)
def _count_kernel(dst3_hbm, onerows_hbm, zrows_hbm, deg_hbm,
                  didx, ones_v, deg_sp, sem):
    c = lax.axis_index("c")
    s = lax.axis_index("s")
    row0 = s * DROWS_PER_SUB
    wid = c * NSUB + s
    pltpu.sync_copy(onerows_hbm, ones_v)
    pltpu.sync_copy(dst3_hbm.at[wid], didx)
    pltpu.sync_copy(zrows_hbm, deg_sp.at[pl.ds(row0, DROWS_PER_SUB)])
    plsc.subcore_barrier()

    def body(k, carry):
        p = lax.rem(k, 2)

        @pl.when(k >= 2)
        def _():
            pltpu.make_async_copy(ones_v, deg_sp.at[didx.at[k - 2]],
                                  sem.at[p]).wait()

        pltpu.async_copy(ones_v, deg_sp.at[didx.at[k]], sem.at[p], add=True)
        return carry

    lax.fori_loop(0, NCHUNK, body, 0)
    pltpu.make_async_copy(ones_v, deg_sp.at[didx.at[NCHUNK - 2]],
                          sem.at[(NCHUNK - 2) % 2]).wait()
    pltpu.make_async_copy(ones_v, deg_sp.at[didx.at[NCHUNK - 1]],
                          sem.at[(NCHUNK - 1) % 2]).wait()
    plsc.subcore_barrier()
    pltpu.sync_copy(deg_sp.at[pl.ds(row0, DROWS_PER_SUB)],
                    deg_hbm.at[c].at[pl.ds(row0, DROWS_PER_SUB)])


@functools.partial(
    pl.kernel,
    mesh=_mesh,
    out_type=jax.ShapeDtypeStruct((NCORES, NPAD, D), jnp.float32),
    scratch_types=[
        pltpu.VMEM((3, CHUNK), jnp.int32),
        pltpu.VMEM((3, CHUNK), jnp.int32),
        pltpu.VMEM((2, CHUNK, D), jnp.float32),
        pltpu.VMEM_SHARED((NPAD, D), jnp.float32),
        pltpu.SemaphoreType.DMA((3,)),
        pltpu.SemaphoreType.DMA((2,)),
        pltpu.SemaphoreType.DMA((2,)),
    ],
)
def _mp_kernel(y_hbm, src_hbm, dst_hbm, zrows_hbm, agg_hbm,
               sidx, didx, rows_v, agg_sp, isem, gsem, ssem):
    c = lax.axis_index("c")
    s = lax.axis_index("s")
    row0 = s * ROWS_PER_SUB
    base = (c * NSUB + s) * EDGES_PER_TILE
    pltpu.sync_copy(zrows_hbm, agg_sp.at[pl.ds(row0, ROWS_PER_SUB)])

    def idx_load(k, j):
        off = pl.multiple_of(base + k * CHUNK, 8)
        pltpu.async_copy(src_hbm.at[pl.ds(off, CHUNK)], sidx.at[j], isem.at[j])
        pltpu.async_copy(dst_hbm.at[pl.ds(off, CHUNK)], didx.at[j], isem.at[j])

    def idx_wait(j):
        pltpu.make_async_copy(src_hbm.at[pl.ds(0, CHUNK)], sidx.at[j],
                              isem.at[j]).wait()
        pltpu.make_async_copy(dst_hbm.at[pl.ds(0, CHUNK)], didx.at[j],
                              isem.at[j]).wait()

    def gather_start(k, p):
        pltpu.async_copy(y_hbm.at[sidx.at[lax.rem(k, 3)]], rows_v.at[p],
                         gsem.at[p])

    def scatter_start(k, p):
        pltpu.async_copy(rows_v.at[p], agg_sp.at[didx.at[lax.rem(k, 3)]],
                         ssem.at[p], add=True)

    def scatter_wait(k, p):
        pltpu.make_async_copy(rows_v.at[p], agg_sp.at[didx.at[lax.rem(k, 3)]],
                              ssem.at[p]).wait()

    idx_load(0, 0)
    idx_load(1, 1)
    plsc.subcore_barrier()
    idx_wait(0)
    gather_start(0, 0)

    def body(k, carry):
        p = lax.rem(k, 2)

        @pl.when(k >= 1)
        def _():
            scatter_wait(k - 1, 1 - p)

        @pl.when(k + 2 < NCHUNK)
        def _():
            idx_load(k + 2, lax.rem(k + 2, 3))

        @pl.when(k + 1 < NCHUNK)
        def _():
            idx_wait(lax.rem(k + 1, 3))
            gather_start(k + 1, 1 - p)

        pltpu.make_async_copy(y_hbm.at[sidx.at[lax.rem(k, 3)]], rows_v.at[p],
                              gsem.at[p]).wait()
        scatter_start(k, p)
        return carry

    lax.fori_loop(0, NCHUNK, body, 0)
    scatter_wait(NCHUNK - 1, (NCHUNK - 1) % 2)
    plsc.subcore_barrier()
    pltpu.sync_copy(agg_sp.at[pl.ds(row0, ROWS_PER_SUB)],
                    agg_hbm.at[c].at[pl.ds(row0, ROWS_PER_SUB)])


ROWS_BLK = 1000
GRID = N // ROWS_BLK


def _tc1_body(x_ref, w_ref, dinv_ref, y_ref):
    xw = jnp.dot(x_ref[...], w_ref[...], preferred_element_type=jnp.float32)
    y_ref[...] = xw * dinv_ref[...]


def _tc2_body(a_ref, y1_ref, dinv_ref, b_ref, w_ref, y2_ref):
    di = dinv_ref[...]
    h = jnp.maximum(di * (a_ref[0] + a_ref[1] + y1_ref[...]) + b_ref[...], 0.0)
    y2_ref[...] = di * jnp.dot(h, w_ref[...], preferred_element_type=jnp.float32)


def _tc3_body(a_ref, y2_ref, dinv_ref, b_ref, batch_ref, wl_ref,
              bl_ref, out_ref, psum_ref):
    i = pl.program_id(0)

    @pl.when(i == 0)
    def _():
        psum_ref[...] = jnp.zeros_like(psum_ref)

    di = dinv_ref[...]
    h = jnp.maximum(di * (a_ref[0] + a_ref[1] + y2_ref[...]) + b_ref[...], 0.0)
    gid = lax.broadcasted_iota(jnp.int32, (1, NUM_GRAPHS), 1)
    onehot = (batch_ref[...] == gid).astype(jnp.float32)
    hcat = jnp.concatenate([h, jnp.ones_like(h)], axis=1)
    psum_ref[...] += lax.dot_general(
        onehot, hcat, (((0,), (0,)), ((), ())),
        preferred_element_type=jnp.float32)

    @pl.when(i == GRID - 1)
    def _():
        cnt = jnp.maximum(psum_ref[:, D:D + 1], 1.0)
        pooled = psum_ref[:, :D] / cnt
        out_ref[...] = (jnp.dot(pooled, wl_ref[...],
                                preferred_element_type=jnp.float32)
                        + bl_ref[...])


def _row_spec(shape_tail):
    nz = len(shape_tail)
    return pl.BlockSpec((ROWS_BLK,) + shape_tail,
                        lambda i, _nz=nz: (i,) + (0,) * _nz)


_pair_spec = pl.BlockSpec((NCORES, ROWS_BLK, D), lambda i: (0, i, 0))


def _full_spec(shape):
    nz = len(shape)
    return pl.BlockSpec(shape, lambda i, _nz=nz: (0,) * _nz)


_tc1 = pl.pallas_call(
    _tc1_body,
    grid=(GRID,),
    in_specs=[_row_spec((D,)), _full_spec((D, D)), _row_spec((1,))],
    out_specs=_row_spec((D,)),
    out_shape=jax.ShapeDtypeStruct((N, D), jnp.float32),
)

_tc2 = pl.pallas_call(
    _tc2_body,
    grid=(GRID,),
    in_specs=[_pair_spec, _row_spec((D,)),
              _row_spec((1,)), _full_spec((1, D)), _full_spec((D, D))],
    out_specs=_row_spec((D,)),
    out_shape=jax.ShapeDtypeStruct((N, D), jnp.float32),
)

_tc3 = pl.pallas_call(
    _tc3_body,
    grid=(GRID,),
    in_specs=[_pair_spec, _row_spec((D,)),
              _row_spec((1,)), _full_spec((1, D)), _row_spec((1,)),
              _full_spec((D, NUM_GRAPHS)), _full_spec((1, NUM_GRAPHS))],
    out_specs=_full_spec((NUM_GRAPHS, NUM_GRAPHS)),
    out_shape=jax.ShapeDtypeStruct((NUM_GRAPHS, NUM_GRAPHS), jnp.float32),
    scratch_shapes=[pltpu.VMEM((NUM_GRAPHS, 2 * D), jnp.float32)],
)


def kernel(x, edge_index, batch, W1, b1, W2, b2, Wl, bl):
    src = edge_index[0]
    dst = edge_index[1]
    dst3 = dst.reshape(NTILES, NCHUNK, CHUNK)

    onerows = jnp.zeros((CHUNK, DEGW), jnp.float32).at[:, 0].set(1.0)
    zrows_deg = jnp.zeros((DROWS_PER_SUB, DEGW), jnp.float32)
    zrows_agg = jnp.zeros((ROWS_PER_SUB, D), jnp.float32)

    deg_pair = _count_kernel(dst3, onerows, zrows_deg)
    deg = deg_pair[0, :N, 0] + deg_pair[1, :N, 0] + 1.0
    dinv = lax.rsqrt(deg).reshape(N, 1)

    y1 = _tc1(x, W1, dinv)
    a1 = _mp_kernel(y1, src, dst, zrows_agg)
    y2 = _tc2(a1, y1, dinv, b1.reshape(1, D), W2)
    a2 = _mp_kernel(y2, src, dst, zrows_agg)
    out = _tc3(a2, y2, dinv, b2.reshape(1, D),
               batch.reshape(N, 1), Wl, bl.reshape(1, NUM_GRAPHS))
    return out


# speedup vs baseline: 26.8902x; 1.0105x over previous
import functools

import jax
import jax.numpy as jnp
from jax import lax
from jax.experimental import pallas as pl
from jax.experimental.pallas import tpu as pltpu
from jax.experimental.pallas import tpu_sc as plsc

N = 10000
E = 320000
D = 128
NUM_GRAPHS = 64

NCORES = 2
NSUB = 16
NTILES = NCORES * NSUB
EDGES_PER_TILE = E // NTILES
CHUNK = 80
NCHUNK = EDGES_PER_TILE // CHUNK
NPAD = 10240
ROWS_PER_SUB = NPAD // NSUB
DEGW = 128
DROWS_PER_SUB = NPAD // NSUB

_mesh = plsc.VectorSubcoreMesh(core_axis_name="c", subcore_axis_name="s")


@functools.partial(
    pl.kernel,
    mesh=_mesh,
    out_type=jax.ShapeDtypeStruct((NCORES, NPAD, DEGW), jnp.float32),
    scratch_types=[
        pltpu.VMEM((NCHUNK, CHUNK), jnp.int32),
        pltpu.VMEM((CHUNK, DEGW), jnp.float32),
        pltpu.VMEM_SHARED((NPAD, DEGW), jnp.float32),
        pltpu.SemaphoreType.DMA((2,)),
    ],
)
def _count_kernel(dst3_hbm, onerows_hbm, zrows_hbm, deg_hbm,
                  didx, ones_v, deg_sp, sem):
    c = lax.axis_index("c")
    s = lax.axis_index("s")
    row0 = s * DROWS_PER_SUB
    wid = c * NSUB + s
    pltpu.sync_copy(onerows_hbm, ones_v)
    pltpu.sync_copy(dst3_hbm.at[wid], didx)
    pltpu.sync_copy(zrows_hbm, deg_sp.at[pl.ds(row0, DROWS_PER_SUB)])
    plsc.subcore_barrier()

    def body(k, carry):
        p = lax.rem(k, 2)

        @pl.when(k >= 2)
        def _():
            pltpu.make_async_copy(ones_v, deg_sp.at[didx.at[k - 2]],
                                  sem.at[p]).wait()

        pltpu.async_copy(ones_v, deg_sp.at[didx.at[k]], sem.at[p], add=True)
        return carry

    lax.fori_loop(0, NCHUNK, body, 0)
    pltpu.make_async_copy(ones_v, deg_sp.at[didx.at[NCHUNK - 2]],
                          sem.at[(NCHUNK - 2) % 2]).wait()
    pltpu.make_async_copy(ones_v, deg_sp.at[didx.at[NCHUNK - 1]],
                          sem.at[(NCHUNK - 1) % 2]).wait()
    plsc.subcore_barrier()
    pltpu.sync_copy(deg_sp.at[pl.ds(row0, DROWS_PER_SUB)],
                    deg_hbm.at[c].at[pl.ds(row0, DROWS_PER_SUB)])


@functools.partial(
    pl.kernel,
    mesh=_mesh,
    out_type=jax.ShapeDtypeStruct((NCORES, NPAD, D), jnp.float32),
    scratch_types=[
        pltpu.VMEM((3, CHUNK), jnp.int32),
        pltpu.VMEM((3, CHUNK), jnp.int32),
        pltpu.VMEM((2, CHUNK, D), jnp.float32),
        pltpu.VMEM_SHARED((NPAD, D), jnp.float32),
        pltpu.SemaphoreType.DMA((3,)),
        pltpu.SemaphoreType.DMA((2,)),
        pltpu.SemaphoreType.DMA((2,)),
    ],
)
def _mp_kernel(y_hbm, src_hbm, dst_hbm, zrows_hbm, agg_hbm,
               sidx, didx, rows_v, agg_sp, isem, gsem, ssem):
    c = lax.axis_index("c")
    s = lax.axis_index("s")
    row0 = s * ROWS_PER_SUB
    base = (c * NSUB + s) * EDGES_PER_TILE
    pltpu.sync_copy(zrows_hbm, agg_sp.at[pl.ds(row0, ROWS_PER_SUB)])

    def idx_load(k, j):
        off = pl.multiple_of(base + k * CHUNK, 8)
        pltpu.async_copy(src_hbm.at[pl.ds(off, CHUNK)], sidx.at[j], isem.at[j])
        pltpu.async_copy(dst_hbm.at[pl.ds(off, CHUNK)], didx.at[j], isem.at[j])

    def idx_wait(j):
        pltpu.make_async_copy(src_hbm.at[pl.ds(0, CHUNK)], sidx.at[j],
                              isem.at[j]).wait()
        pltpu.make_async_copy(dst_hbm.at[pl.ds(0, CHUNK)], didx.at[j],
                              isem.at[j]).wait()

    def gather_start(k, p):
        pltpu.async_copy(y_hbm.at[sidx.at[lax.rem(k, 3)]], rows_v.at[p],
                         gsem.at[p])

    def scatter_start(k, p):
        pltpu.async_copy(rows_v.at[p], agg_sp.at[didx.at[lax.rem(k, 3)]],
                         ssem.at[p], add=True)

    def scatter_wait(k, p):
        pltpu.make_async_copy(rows_v.at[p], agg_sp.at[didx.at[lax.rem(k, 3)]],
                              ssem.at[p]).wait()

    idx_load(0, 0)
    idx_load(1, 1)
    plsc.subcore_barrier()
    idx_wait(0)
    gather_start(0, 0)

    def body(k, carry):
        p = lax.rem(k, 2)

        @pl.when(k >= 1)
        def _():
            scatter_wait(k - 1, 1 - p)

        @pl.when(k + 2 < NCHUNK)
        def _():
            idx_load(k + 2, lax.rem(k + 2, 3))

        @pl.when(k + 1 < NCHUNK)
        def _():
            idx_wait(lax.rem(k + 1, 3))
            gather_start(k + 1, 1 - p)

        pltpu.make_async_copy(y_hbm.at[sidx.at[lax.rem(k, 3)]], rows_v.at[p],
                              gsem.at[p]).wait()
        scatter_start(k, p)
        return carry

    lax.fori_loop(0, NCHUNK, body, 0)
    scatter_wait(NCHUNK - 1, (NCHUNK - 1) % 2)
    plsc.subcore_barrier()
    pltpu.sync_copy(agg_sp.at[pl.ds(row0, ROWS_PER_SUB)],
                    agg_hbm.at[c].at[pl.ds(row0, ROWS_PER_SUB)])


ROWS_BLK = 1000
GRID = N // ROWS_BLK


def _tc1_body(x_ref, w_ref, deg_ref, y_ref, dinv_ref):
    deg = deg_ref[0, :, 0:1] + deg_ref[1, :, 0:1] + 1.0
    di = lax.rsqrt(deg)
    xw = jnp.dot(x_ref[...], w_ref[...], preferred_element_type=jnp.float32)
    y_ref[...] = xw * di
    dinv_ref[...] = di


def _tc2_body(a_ref, y1_ref, dinv_ref, b_ref, w_ref, y2_ref):
    di = dinv_ref[...]
    h = jnp.maximum(di * (a_ref[0] + a_ref[1] + y1_ref[...]) + b_ref[...], 0.0)
    y2_ref[...] = di * jnp.dot(h, w_ref[...], preferred_element_type=jnp.float32)


def _tc3_body(a_ref, y2_ref, dinv_ref, b_ref, batch_ref, wl_ref,
              bl_ref, out_ref, psum_ref):
    i = pl.program_id(0)

    @pl.when(i == 0)
    def _():
        psum_ref[...] = jnp.zeros_like(psum_ref)

    di = dinv_ref[...]
    h = jnp.maximum(di * (a_ref[0] + a_ref[1] + y2_ref[...]) + b_ref[...], 0.0)
    gid = lax.broadcasted_iota(jnp.int32, (1, NUM_GRAPHS), 1)
    onehot = (batch_ref[...] == gid).astype(jnp.float32)
    hcat = jnp.concatenate([h, jnp.ones_like(h)], axis=1)
    psum_ref[...] += lax.dot_general(
        onehot, hcat, (((0,), (0,)), ((), ())),
        preferred_element_type=jnp.float32)

    @pl.when(i == GRID - 1)
    def _():
        cnt = jnp.maximum(psum_ref[:, D:D + 1], 1.0)
        pooled = psum_ref[:, :D] / cnt
        out_ref[...] = (jnp.dot(pooled, wl_ref[...],
                                preferred_element_type=jnp.float32)
                        + bl_ref[...])


def _row_spec(shape_tail):
    nz = len(shape_tail)
    return pl.BlockSpec((ROWS_BLK,) + shape_tail,
                        lambda i, _nz=nz: (i,) + (0,) * _nz)


_pair_spec = pl.BlockSpec((NCORES, ROWS_BLK, D), lambda i: (0, i, 0))


def _full_spec(shape):
    nz = len(shape)
    return pl.BlockSpec(shape, lambda i, _nz=nz: (0,) * _nz)


_tc1 = pl.pallas_call(
    _tc1_body,
    grid=(GRID,),
    in_specs=[_row_spec((D,)), _full_spec((D, D)), _pair_spec],
    out_specs=(_row_spec((D,)), _row_spec((1,))),
    out_shape=(jax.ShapeDtypeStruct((N, D), jnp.float32),
               jax.ShapeDtypeStruct((N, 1), jnp.float32)),
)

_tc2 = pl.pallas_call(
    _tc2_body,
    grid=(GRID,),
    in_specs=[_pair_spec, _row_spec((D,)),
              _row_spec((1,)), _full_spec((1, D)), _full_spec((D, D))],
    out_specs=_row_spec((D,)),
    out_shape=jax.ShapeDtypeStruct((N, D), jnp.float32),
)

_tc3 = pl.pallas_call(
    _tc3_body,
    grid=(GRID,),
    in_specs=[_pair_spec, _row_spec((D,)),
              _row_spec((1,)), _full_spec((1, D)), _row_spec((1,)),
              _full_spec((D, NUM_GRAPHS)), _full_spec((1, NUM_GRAPHS))],
    out_specs=_full_spec((NUM_GRAPHS, NUM_GRAPHS)),
    out_shape=jax.ShapeDtypeStruct((NUM_GRAPHS, NUM_GRAPHS), jnp.float32),
    scratch_shapes=[pltpu.VMEM((NUM_GRAPHS, 2 * D), jnp.float32)],
)


def kernel(x, edge_index, batch, W1, b1, W2, b2, Wl, bl):
    src = edge_index[0]
    dst = edge_index[1]
    dst3 = dst.reshape(NTILES, NCHUNK, CHUNK)

    onerows = jnp.zeros((CHUNK, DEGW), jnp.float32).at[:, 0].set(1.0)
    zrows_deg = jnp.zeros((DROWS_PER_SUB, DEGW), jnp.float32)
    zrows_agg = jnp.zeros((ROWS_PER_SUB, D), jnp.float32)

    deg_pair = _count_kernel(dst3, onerows, zrows_deg)
    y1, dinv = _tc1(x, W1, deg_pair)
    a1 = _mp_kernel(y1, src, dst, zrows_agg)
    y2 = _tc2(a1, y1, dinv, b1.reshape(1, D), W2)
    a2 = _mp_kernel(y2, src, dst, zrows_agg)
    out = _tc3(a2, y2, dinv, b2.reshape(1, D),
               batch.reshape(N, 1), Wl, bl.reshape(1, NUM_GRAPHS))
    return out
